# baseline (speedup 1.0000x reference)
import jax
import jax.numpy as jnp
from jax.experimental import pallas as pl

_HIDDEN = 2048
_HALF = _HIDDEN // 2
_NE = 8


def _router_body(xa_ref, xb_ref, w1_ref, b1_ref, w2_ref, b2_ref,
                 la_ref, lb_ref, ia_ref, ib_ref):
    for x_ref, l_ref, i_ref in ((xa_ref, la_ref, ia_ref),
                                (xb_ref, lb_ref, ib_ref)):
        h = jnp.dot(x_ref[:], w1_ref[:], preferred_element_type=jnp.float32)
        h = jnp.maximum(h + b1_ref[:], 0.0)
        logits = jnp.dot(h, w2_ref[:], preferred_element_type=jnp.float32)
        logits = logits + b2_ref[:]
        l_ref[:] = logits
        lt = logits.T
        m = jnp.max(lt, axis=0, keepdims=True)
        expert = jax.lax.broadcasted_iota(jnp.int32, lt.shape, 0)
        i_ref[:] = jnp.min(jnp.where(lt == m, expert, _NE), axis=0)


def kernel(hidden_states, W1, b1, W2, b2):
    B, S, H = hidden_states.shape
    M = B * S
    half_m = M // 2
    x = hidden_states.reshape(M, H)
    m_blk = 512
    n_steps = half_m // m_blk
    grid = (n_steps,)

    la, lb, ia, ib = pl.pallas_call(
        _router_body,
        grid=grid,
        in_specs=[
            pl.BlockSpec((m_blk, H), lambda i: (i, 0)),
            pl.BlockSpec((m_blk, H), lambda i: (i + n_steps, 0)),
            pl.BlockSpec((H, _HALF), lambda i: (0, 0)),
            pl.BlockSpec((1, _HALF), lambda i: (0, 0)),
            pl.BlockSpec((_HALF, _NE), lambda i: (0, 0)),
            pl.BlockSpec((1, _NE), lambda i: (0, 0)),
        ],
        out_specs=[
            pl.BlockSpec((m_blk, _NE), lambda i: (i, 0)),
            pl.BlockSpec((m_blk, _NE), lambda i: (i, 0)),
            pl.BlockSpec((m_blk,), lambda i: (i,)),
            pl.BlockSpec((m_blk,), lambda i: (i,)),
        ],
        out_shape=[
            jax.ShapeDtypeStruct((half_m, _NE), jnp.float32),
            jax.ShapeDtypeStruct((half_m, _NE), jnp.float32),
            jax.ShapeDtypeStruct((half_m,), jnp.int32),
            jax.ShapeDtypeStruct((half_m,), jnp.int32),
        ],
    )(x, x, W1, b1.reshape(1, _HALF), W2, b2.reshape(1, _NE))

    logits = jnp.concatenate([la, lb], axis=0).reshape(B, S, _NE)
    idx = jnp.concatenate([ia, ib], axis=0).reshape(B, S)
    return idx, logits


# speedup vs baseline: 1.0773x; 1.0773x over previous
import jax
import jax.numpy as jnp
from jax.experimental import pallas as pl

_HIDDEN = 2048
_HALF = _HIDDEN // 2
_NE = 8


def _router_body(x0_ref, x1_ref, x2_ref, x3_ref, w1_ref, b1_ref, w2_ref,
                 b2_ref, lt_ref, idx_ref):
    for b, x_ref in enumerate((x0_ref, x1_ref, x2_ref, x3_ref)):
        h = jnp.dot(x_ref[:], w1_ref[:], preferred_element_type=jnp.float32)
        h = jnp.maximum(h + b1_ref[:], 0.0)
        logits = jnp.dot(h, w2_ref[:], preferred_element_type=jnp.float32)
        lt = logits.T + b2_ref[:]
        lt_ref[b] = lt
        m = jnp.max(lt, axis=0, keepdims=True)
        expert = jax.lax.broadcasted_iota(jnp.int32, lt.shape, 0)
        idx_ref[b] = jnp.min(jnp.where(lt == m, expert, _NE), axis=0)


def kernel(hidden_states, W1, b1, W2, b2):
    B, S, H = hidden_states.shape
    M = B * S
    x = hidden_states.reshape(M, H)
    m_blk = 512
    n_steps = S // m_blk
    grid = (n_steps,)

    def x_spec(b):
        return pl.BlockSpec((m_blk, H), lambda j, b=b: (b * n_steps + j, 0))

    lt, idx = pl.pallas_call(
        _router_body,
        grid=grid,
        in_specs=[
            x_spec(0),
            x_spec(1),
            x_spec(2),
            x_spec(3),
            pl.BlockSpec((H, _HALF), lambda j: (0, 0)),
            pl.BlockSpec((1, _HALF), lambda j: (0, 0)),
            pl.BlockSpec((_HALF, _NE), lambda j: (0, 0)),
            pl.BlockSpec((_NE, 1), lambda j: (0, 0)),
        ],
        out_specs=[
            pl.BlockSpec((B, _NE, m_blk), lambda j: (0, 0, j)),
            pl.BlockSpec((B, m_blk), lambda j: (0, j)),
        ],
        out_shape=[
            jax.ShapeDtypeStruct((B, _NE, S), jnp.float32),
            jax.ShapeDtypeStruct((B, S), jnp.int32),
        ],
    )(x, x, x, x, W1, b1.reshape(1, _HALF), W2, b2.reshape(_NE, 1))

    return idx, jnp.transpose(lt, (0, 2, 1))


# speedup vs baseline: 1.1177x; 1.0375x over previous
import jax
import jax.numpy as jnp
from jax.experimental import pallas as pl

_HIDDEN = 2048
_HALF = _HIDDEN // 2
_NE = 8


def _router_body(x0_ref, x1_ref, x2_ref, x3_ref, w1_ref, b1_ref, w2_ref,
                 b2_ref, lt_ref, idx_ref):
    for b, x_ref in enumerate((x0_ref, x1_ref, x2_ref, x3_ref)):
        h = jnp.dot(x_ref[:], w1_ref[:], preferred_element_type=jnp.float32)
        h = jnp.maximum(h + b1_ref[:], 0.0)
        lt = jax.lax.dot_general(
            w2_ref[:], h, (((1,), (1,)), ((), ())),
            preferred_element_type=jnp.float32,
        ) + b2_ref[:]
        lt_ref[b] = lt
        m = jnp.max(lt, axis=0, keepdims=True)
        expert = jax.lax.broadcasted_iota(jnp.int32, lt.shape, 0)
        idx_ref[b] = jnp.min(jnp.where(lt == m, expert, _NE), axis=0)


def kernel(hidden_states, W1, b1, W2, b2):
    B, S, H = hidden_states.shape
    M = B * S
    x = hidden_states.reshape(M, H)
    m_blk = 512
    n_steps = S // m_blk
    grid = (n_steps,)

    def x_spec(b):
        return pl.BlockSpec((m_blk, H), lambda j, b=b: (b * n_steps + j, 0))

    lt, idx = pl.pallas_call(
        _router_body,
        grid=grid,
        in_specs=[
            x_spec(0),
            x_spec(1),
            x_spec(2),
            x_spec(3),
            pl.BlockSpec((H, _HALF), lambda j: (0, 0)),
            pl.BlockSpec((1, _HALF), lambda j: (0, 0)),
            pl.BlockSpec((_NE, _HALF), lambda j: (0, 0)),
            pl.BlockSpec((_NE, 1), lambda j: (0, 0)),
        ],
        out_specs=[
            pl.BlockSpec((B, _NE, m_blk), lambda j: (0, 0, j)),
            pl.BlockSpec((B, m_blk), lambda j: (0, j)),
        ],
        out_shape=[
            jax.ShapeDtypeStruct((B, _NE, S), jnp.float32),
            jax.ShapeDtypeStruct((B, S), jnp.int32),
        ],
    )(x, x, x, x, W1, b1.reshape(1, _HALF), W2.T, b2.reshape(_NE, 1))

    return idx, jnp.transpose(lt, (0, 2, 1))
